# baseline (speedup 1.0000x reference)
import functools

import jax
import jax.numpy as jnp
from jax import lax
from jax.experimental import pallas as pl
from jax.experimental.pallas import tpu as pltpu
from jax.experimental.pallas import tpu_sc as plsc

_OHEM_RATIO = 0.7
_EPS = 1e-07

_NW = 32
_LANES = 16
_NG = 2
_NB_A = 1024
_NB_B = 2048
_NB_C = 1024


def _loss_map_body(pred_ref, target_ref, out_ref):
    x = pred_ref[0]
    t = target_ref[0]
    m = jnp.max(x, axis=0)
    s = jnp.sum(jnp.exp(x - m[None]), axis=0)
    tsum = jnp.sum(t, axis=0)
    dot = jnp.sum(t * x, axis=0)
    loss = tsum * (m + jnp.log(s)) - dot
    out_ref[0] = jnp.maximum(loss, 0.0)


def _lane_iota():
    return lax.iota(jnp.int32, _LANES)


def _scalar_at(v, i):
    return jnp.sum(jnp.where(_lane_iota() == i, v, jnp.zeros_like(v)))


def _zero_hist(hist_ref, nwords):
    z = jnp.zeros((_LANES,), jnp.int32)

    def body(j, c):
        for u in range(8):
            hist_ref[pl.ds((j * 8 + u) * _LANES, _LANES)] = z
        return c

    lax.fori_loop(0, nwords // (8 * _LANES), body, 0)


def _lane_reduce(hist_ref, out_ref, nb):
    def body(j, c):
        acc = hist_ref[pl.ds(j * _LANES, _LANES)]
        for l in range(1, _LANES):
            acc = acc + hist_ref[pl.ds(l * (nb + 17) + j * _LANES, _LANES)]
        out_ref[pl.ds(j * _LANES, _LANES)] = acc
        return c

    lax.fori_loop(0, nb // _LANES, body, 0)


def _merge_find(hbuf_ref, nrows, nb, r):
    ngroups = nb // _LANES

    def body(jj, carry):
        found, bstar, r_rem, acc = carry
        j = ngroups - 1 - jj
        h = hbuf_ref[0, pl.ds(j * _LANES, _LANES)]
        for t in range(1, nrows):
            h = h + hbuf_ref[t, pl.ds(j * _LANES, _LANES)]
        rev = lax.rev(h, (0,))
        cs = plsc.cumsum(rev) + acc
        mask = cs >= r
        pc = jnp.max(plsc.all_reduce_population_count(mask))
        tstar = jnp.max(plsc.all_reduce_ffs(mask))
        hit = jnp.logical_and(pc > 0, jnp.logical_not(found))
        b_new = j * _LANES + (_LANES - 1) - tstar
        r_new = r - (_scalar_at(cs, tstar) - _scalar_at(rev, tstar))
        bstar = jnp.where(hit, b_new, bstar)
        r_rem = jnp.where(hit, r_new, r_rem)
        found = jnp.logical_or(found, pc > 0)
        acc = _scalar_at(cs, _LANES - 1)
        return found, bstar, r_rem, acc

    _, bstar, r_rem, _ = lax.fori_loop(
        0, ngroups, body,
        (jnp.bool_(False), jnp.int32(0), jnp.int32(1), jnp.int32(0)))
    return bstar, r_rem


def _wid():
    return lax.axis_index("s") * 2 + lax.axis_index("c")


def _sc_pass_a(bits_hbm, h1_hbm, chunk, hist, hrow):
    wid = _wid()
    pltpu.sync_copy(bits_hbm.at[wid], chunk)
    _zero_hist(hist, (_NB_A + 16) * _LANES)
    lane = _lane_iota()
    ones = jnp.ones((_LANES,), jnp.int32)
    nit = chunk.shape[0] // _LANES
    lane_nb = lane * (_NB_A + 17)
    U = 8

    def body(i, c):
        for u in range(U):
            bits = chunk[pl.ds((i * U + u) * _LANES, _LANES)]
            idx = lane_nb + jnp.right_shift(bits, 21)
            plsc.addupdate_scatter(hist, [idx], ones)
        return c

    lax.fori_loop(0, nit // U, body, 0)
    _lane_reduce(hist, hrow, _NB_A)
    pltpu.sync_copy(hrow, h1_hbm.at[wid])


def _sc_pass_b(bits_hbm, h1_hbm, h2_hbm, rec1_hbm, hbuf, chunk, hist, hrow,
               recbuf, *, k):
    wid = _wid()
    pltpu.sync_copy(h1_hbm, hbuf)
    pa, ra = _merge_find(hbuf, _NW, _NB_A, jnp.int32(k))

    _zero_hist(hist, (_NB_B + 16) * _LANES)
    lane = _lane_iota()
    lane_nb = lane * (_NB_B + 17)
    ones = jnp.ones((_LANES,), jnp.int32)
    npieces = 4
    piece = bits_hbm.shape[1] // npieces
    nit = piece // _LANES

    def piece_body(p, c):
        pltpu.sync_copy(bits_hbm.at[wid, pl.ds(p * piece, piece)], chunk)

        def body(i, c2):
            for u in range(8):
                bits = chunk[pl.ds((i * 8 + u) * _LANES, _LANES)]
                sel = jnp.right_shift(bits, 21) == pa
                idx = lane_nb + jnp.bitwise_and(jnp.right_shift(bits, 10),
                                                _NB_B - 1)
                plsc.addupdate_scatter(hist, [idx], ones, mask=sel)
            return c2

        lax.fori_loop(0, nit // 8, body, 0)
        return c

    lax.fori_loop(0, npieces, piece_body, 0)
    _lane_reduce(hist, hrow, _NB_B)
    pltpu.sync_copy(hrow, h2_hbm.at[wid])

    @pl.when(wid == 0)
    def _():
        rec = jnp.where(lane == 0, pa, jnp.where(lane == 1, ra, 0))
        recbuf[...] = rec
        pltpu.sync_copy(recbuf, rec1_hbm)


def _sc_pass_c(bits_hbm, h2_hbm, rec1_hbm, h3_hbm, rec2_hbm, hbuf, chunk,
               hist, hrow, recbuf):
    wid = _wid()
    pltpu.sync_copy(rec1_hbm, recbuf)
    rec1 = recbuf[...]
    pa = _scalar_at(rec1, 0)
    ra = _scalar_at(rec1, 1)

    pltpu.sync_copy(h2_hbm, hbuf)
    pb, rb = _merge_find(hbuf, _NW, _NB_B, ra)
    pab = jnp.bitwise_or(lax.shift_left(pa, 11), pb)

    _zero_hist(hist, (_NB_C + 16) * _LANES)
    lane = _lane_iota()
    lane_nb = lane * (_NB_C + 17)
    ones = jnp.ones((_LANES,), jnp.int32)
    npieces = 4
    piece = bits_hbm.shape[1] // npieces
    nit = piece // _LANES

    def piece_body(p, c):
        pltpu.sync_copy(bits_hbm.at[wid, pl.ds(p * piece, piece)], chunk)

        def body(i, c2):
            for u in range(8):
                bits = chunk[pl.ds((i * 8 + u) * _LANES, _LANES)]
                sel = jnp.right_shift(bits, 10) == pab
                idx = lane_nb + jnp.bitwise_and(bits, _NB_C - 1)
                plsc.addupdate_scatter(hist, [idx], ones, mask=sel)
            return c2

        lax.fori_loop(0, nit // 8, body, 0)
        return c

    lax.fori_loop(0, npieces, piece_body, 0)
    _lane_reduce(hist, hrow, _NB_C)
    pltpu.sync_copy(hrow, h3_hbm.at[wid])

    @pl.when(wid == 0)
    def _():
        rec = jnp.where(lane == 0, pab, jnp.where(lane == 1, rb, 0))
        recbuf[...] = rec
        pltpu.sync_copy(recbuf, rec2_hbm)


def _finish_body(loss_ref, h2_ref, rec1_ref, out_ref, *, k):
    cnt = jnp.sum(h2_ref[...].astype(jnp.float32), axis=0, keepdims=True)
    row = lax.broadcasted_iota(jnp.int32, (_NB_B, _NB_B), 0)
    col = lax.broadcasted_iota(jnp.int32, (_NB_B, _NB_B), 1)
    tri = (row >= col).astype(jnp.float32)
    suf_cnt = jnp.dot(cnt, tri, preferred_element_type=jnp.float32)

    lane16 = lax.broadcasted_iota(jnp.int32, (1, 16), 1)
    rec = rec1_ref[...]
    pa = jnp.sum(jnp.where(lane16 == 0, rec, 0))
    ra = jnp.sum(jnp.where(lane16 == 1, rec, 0)).astype(jnp.float32)

    iota = lax.broadcasted_iota(jnp.int32, (1, _NB_B), 1)
    cond = suf_cnt >= ra
    pb = jnp.max(jnp.where(cond, iota, -1))
    pab = jnp.bitwise_or(lax.shift_left(pa, 11), pb)

    l0 = loss_ref[...]
    b0 = l0.view(jnp.int32)
    base = lax.shift_left(pab, 10)

    def step(i, prefix):
        cand = prefix | (jnp.int32(1) << (jnp.int32(9) - i))
        cnt_ge = jnp.sum((b0 >= cand).astype(jnp.int32))
        return jnp.where(cnt_ge >= k, cand, prefix)

    vbits = jax.lax.fori_loop(0, 10, step, base)
    thresh = jax.lax.bitcast_convert_type(vbits, jnp.float32)

    m0 = l0 >= thresh
    s = jnp.sum(jnp.where(m0, l0, 0.0))
    c = jnp.sum(m0.astype(jnp.int32)).astype(jnp.float32)
    out_ref[...] = jnp.reshape(s / (c + _EPS), (1, 1))


def kernel(pred, target):
    B, C, H, W = pred.shape
    BH = 64
    BG = B // _NG
    mesh = plsc.VectorSubcoreMesh(core_axis_name="c", subcore_axis_name="s")
    scp = pltpu.CompilerParams(needs_layout_passes=False)

    n = B * H * W
    k = int(n * _OHEM_RATIO)
    gchunk = (n // _NG) // _NW

    loss = pl.pallas_call(
        _loss_map_body,
        grid=(B, H // BH),
        in_specs=[
            pl.BlockSpec((1, C, BH, W), lambda b, h: (b, 0, h, 0)),
            pl.BlockSpec((1, C, BH, W), lambda b, h: (b, 0, h, 0)),
        ],
        out_specs=pl.BlockSpec((1, BH, W), lambda b, h: (b, h, 0)),
        out_shape=jax.ShapeDtypeStruct((B, H, W), jnp.float32),
    )(pred, target)
    bits_all = lax.bitcast_convert_type(loss, jnp.int32).reshape(_NW, n // _NW)
    h1 = pl.kernel(
        _sc_pass_a,
        out_type=jax.ShapeDtypeStruct((_NW, _NB_A), jnp.int32),
        mesh=mesh,
        compiler_params=scp,
        scratch_types=[
            pltpu.VMEM((n // _NW,), jnp.int32),
            pltpu.VMEM(((_NB_A + 16) * _LANES,), jnp.int32),
            pltpu.VMEM((_NB_A,), jnp.int32),
        ],
    )(bits_all)

    h2, rec1 = pl.kernel(
        functools.partial(_sc_pass_b, k=k),
        out_type=(
            jax.ShapeDtypeStruct((_NW, _NB_B), jnp.int32),
            jax.ShapeDtypeStruct((_LANES,), jnp.int32),
        ),
        mesh=mesh,
        compiler_params=scp,
        scratch_types=[
            pltpu.VMEM((_NW, _NB_A), jnp.int32),
            pltpu.VMEM((n // _NW // 4,), jnp.int32),
            pltpu.VMEM(((_NB_B + 16) * _LANES,), jnp.int32),
            pltpu.VMEM((_NB_B,), jnp.int32),
            pltpu.VMEM((_LANES,), jnp.int32),
        ],
    )(bits_all, h1)

    out = pl.pallas_call(
        functools.partial(_finish_body, k=k),
        out_shape=jax.ShapeDtypeStruct((1, 1), jnp.float32),
    )(loss.reshape(n // 1024, 1024), h2, rec1.reshape(1, _LANES))
    return out[0, 0]


# speedup vs baseline: 1.1320x; 1.1320x over previous
import functools

import jax
import jax.numpy as jnp
from jax import lax
from jax.experimental import pallas as pl
from jax.experimental.pallas import tpu as pltpu
from jax.experimental.pallas import tpu_sc as plsc

_OHEM_RATIO = 0.7
_EPS = 1e-07

_NW = 32
_LANES = 16
_NG = 2
_NB_A = 1024
_NB_B = 2048
_NB_C = 1024


def _loss_map_body(pred_ref, target_ref, out_ref):
    x = pred_ref[0]
    t = target_ref[0]
    m = jnp.max(x, axis=0)
    s = jnp.sum(jnp.exp(x - m[None]), axis=0)
    tsum = jnp.sum(t, axis=0)
    dot = jnp.sum(t * x, axis=0)
    loss = tsum * (m + jnp.log(s)) - dot
    out_ref[0] = jnp.maximum(loss, 0.0)


def _lane_iota():
    return lax.iota(jnp.int32, _LANES)


def _scalar_at(v, i):
    return jnp.sum(jnp.where(_lane_iota() == i, v, jnp.zeros_like(v)))


def _zero_hist(hist_ref, nwords):
    z = jnp.zeros((_LANES,), jnp.int32)

    def body(j, c):
        for u in range(8):
            hist_ref[pl.ds((j * 8 + u) * _LANES, _LANES)] = z
        return c

    lax.fori_loop(0, nwords // (8 * _LANES), body, 0)


def _lane_reduce(hist_ref, out_ref, nb):
    def body(j, c):
        acc = hist_ref[pl.ds(j * _LANES, _LANES)]
        for l in range(1, _LANES):
            acc = acc + hist_ref[pl.ds(l * (nb + 17) + j * _LANES, _LANES)]
        out_ref[pl.ds(j * _LANES, _LANES)] = acc
        return c

    lax.fori_loop(0, nb // _LANES, body, 0)


def _merge_find(hbuf_ref, nrows, nb, r):
    ngroups = nb // _LANES

    def body(jj, carry):
        found, bstar, r_rem, acc = carry
        j = ngroups - 1 - jj
        h = hbuf_ref[0, pl.ds(j * _LANES, _LANES)]
        for t in range(1, nrows):
            h = h + hbuf_ref[t, pl.ds(j * _LANES, _LANES)]
        rev = lax.rev(h, (0,))
        cs = plsc.cumsum(rev) + acc
        mask = cs >= r
        pc = jnp.max(plsc.all_reduce_population_count(mask))
        tstar = jnp.max(plsc.all_reduce_ffs(mask))
        hit = jnp.logical_and(pc > 0, jnp.logical_not(found))
        b_new = j * _LANES + (_LANES - 1) - tstar
        r_new = r - (_scalar_at(cs, tstar) - _scalar_at(rev, tstar))
        bstar = jnp.where(hit, b_new, bstar)
        r_rem = jnp.where(hit, r_new, r_rem)
        found = jnp.logical_or(found, pc > 0)
        acc = _scalar_at(cs, _LANES - 1)
        return found, bstar, r_rem, acc

    _, bstar, r_rem, _ = lax.fori_loop(
        0, ngroups, body,
        (jnp.bool_(False), jnp.int32(0), jnp.int32(1), jnp.int32(0)))
    return bstar, r_rem


def _wid():
    return lax.axis_index("s") * 2 + lax.axis_index("c")


def _sc_pass_a(bits_hbm, h1_hbm, chunk, hist, hrow):
    wid = _wid()
    pltpu.sync_copy(bits_hbm.at[wid], chunk)
    _zero_hist(hist, (_NB_A + 16) * _LANES)
    lane = _lane_iota()
    ones = jnp.ones((_LANES,), jnp.int32)
    nit = chunk.shape[0] // _LANES
    lane_nb = lane * (_NB_A + 17)
    U = 8

    def body(i, c):
        for u in range(U):
            bits = chunk[pl.ds((i * U + u) * _LANES, _LANES)]
            idx = lane_nb + jnp.right_shift(bits, 21)
            plsc.addupdate_scatter(hist, [idx], ones)
        return c

    lax.fori_loop(0, nit // U, body, 0)
    _lane_reduce(hist, hrow, _NB_A)
    pltpu.sync_copy(hrow, h1_hbm.at[wid])


def _sc_pass_b(bits_hbm, h1_hbm, h2_hbm, rec1_hbm, hbuf, chunk, hist, hrow,
               recbuf, *, k):
    wid = _wid()
    pltpu.sync_copy(h1_hbm, hbuf)
    pa, ra = _merge_find(hbuf, _NG * _NW, _NB_A, jnp.int32(k))

    _zero_hist(hist, (_NB_B + 16) * _LANES)
    lane = _lane_iota()
    lane_nb = lane * (_NB_B + 17)
    ones = jnp.ones((_LANES,), jnp.int32)
    npieces = 4
    piece = bits_hbm.shape[1] // npieces
    nit = piece // _LANES

    def piece_body(p, c):
        pltpu.sync_copy(bits_hbm.at[wid, pl.ds(p * piece, piece)], chunk)

        def body(i, c2):
            for u in range(8):
                bits = chunk[pl.ds((i * 8 + u) * _LANES, _LANES)]
                sel = jnp.right_shift(bits, 21) == pa
                idx = lane_nb + jnp.bitwise_and(jnp.right_shift(bits, 10),
                                                _NB_B - 1)
                plsc.addupdate_scatter(hist, [idx], ones, mask=sel)
            return c2

        lax.fori_loop(0, nit // 8, body, 0)
        return c

    lax.fori_loop(0, npieces, piece_body, 0)
    _lane_reduce(hist, hrow, _NB_B)
    pltpu.sync_copy(hrow, h2_hbm.at[wid])

    @pl.when(wid == 0)
    def _():
        rec = jnp.where(lane == 0, pa, jnp.where(lane == 1, ra, 0))
        recbuf[...] = rec
        pltpu.sync_copy(recbuf, rec1_hbm)


def _sc_pass_c(bits_hbm, h2_hbm, rec1_hbm, h3_hbm, rec2_hbm, hbuf, chunk,
               hist, hrow, recbuf):
    wid = _wid()
    pltpu.sync_copy(rec1_hbm, recbuf)
    rec1 = recbuf[...]
    pa = _scalar_at(rec1, 0)
    ra = _scalar_at(rec1, 1)

    pltpu.sync_copy(h2_hbm, hbuf)
    pb, rb = _merge_find(hbuf, _NW, _NB_B, ra)
    pab = jnp.bitwise_or(lax.shift_left(pa, 11), pb)

    _zero_hist(hist, (_NB_C + 16) * _LANES)
    lane = _lane_iota()
    lane_nb = lane * (_NB_C + 17)
    ones = jnp.ones((_LANES,), jnp.int32)
    npieces = 4
    piece = bits_hbm.shape[1] // npieces
    nit = piece // _LANES

    def piece_body(p, c):
        pltpu.sync_copy(bits_hbm.at[wid, pl.ds(p * piece, piece)], chunk)

        def body(i, c2):
            for u in range(8):
                bits = chunk[pl.ds((i * 8 + u) * _LANES, _LANES)]
                sel = jnp.right_shift(bits, 10) == pab
                idx = lane_nb + jnp.bitwise_and(bits, _NB_C - 1)
                plsc.addupdate_scatter(hist, [idx], ones, mask=sel)
            return c2

        lax.fori_loop(0, nit // 8, body, 0)
        return c

    lax.fori_loop(0, npieces, piece_body, 0)
    _lane_reduce(hist, hrow, _NB_C)
    pltpu.sync_copy(hrow, h3_hbm.at[wid])

    @pl.when(wid == 0)
    def _():
        rec = jnp.where(lane == 0, pab, jnp.where(lane == 1, rb, 0))
        recbuf[...] = rec
        pltpu.sync_copy(recbuf, rec2_hbm)


def _finish_body(loss0_ref, loss1_ref, h2_ref, rec1_ref, out_ref, *, k):
    cnt = jnp.sum(h2_ref[...].astype(jnp.float32), axis=0, keepdims=True)
    row = lax.broadcasted_iota(jnp.int32, (_NB_B, _NB_B), 0)
    col = lax.broadcasted_iota(jnp.int32, (_NB_B, _NB_B), 1)
    tri = (row >= col).astype(jnp.float32)
    suf_cnt = jnp.dot(cnt, tri, preferred_element_type=jnp.float32)

    lane16 = lax.broadcasted_iota(jnp.int32, (1, 16), 1)
    rec = rec1_ref[...]
    pa = jnp.sum(jnp.where(lane16 == 0, rec, 0))
    ra = jnp.sum(jnp.where(lane16 == 1, rec, 0)).astype(jnp.float32)

    iota = lax.broadcasted_iota(jnp.int32, (1, _NB_B), 1)
    cond = suf_cnt >= ra
    pb = jnp.max(jnp.where(cond, iota, -1))
    pab = jnp.bitwise_or(lax.shift_left(pa, 11), pb)

    l0 = loss0_ref[...]
    l1 = loss1_ref[...]
    b0 = l0.view(jnp.int32)
    b1 = l1.view(jnp.int32)
    base = lax.shift_left(pab, 10)

    def step(i, prefix):
        cand = prefix | (jnp.int32(1) << (jnp.int32(9) - i))
        cnt_ge = (jnp.sum((b0 >= cand).astype(jnp.int32)) +
                  jnp.sum((b1 >= cand).astype(jnp.int32)))
        return jnp.where(cnt_ge >= k, cand, prefix)

    vbits = jax.lax.fori_loop(0, 10, step, base)
    thresh = jax.lax.bitcast_convert_type(vbits, jnp.float32)

    m0 = l0 >= thresh
    m1 = l1 >= thresh
    s = jnp.sum(jnp.where(m0, l0, 0.0)) + jnp.sum(jnp.where(m1, l1, 0.0))
    c = (jnp.sum(m0.astype(jnp.int32)) +
         jnp.sum(m1.astype(jnp.int32))).astype(jnp.float32)
    out_ref[...] = jnp.reshape(s / (c + _EPS), (1, 1))


def kernel(pred, target):
    B, C, H, W = pred.shape
    BH = 128
    BG = B // _NG
    mesh = plsc.VectorSubcoreMesh(core_axis_name="c", subcore_axis_name="s")
    scp = pltpu.CompilerParams(needs_layout_passes=False)

    n = B * H * W
    k = int(n * _OHEM_RATIO)
    gchunk = (n // _NG) // _NW

    loss_g = []
    h1_g = []
    bits_g = []
    for g in range(_NG):
        loss = pl.pallas_call(
            _loss_map_body,
            grid=(BG, H // BH),
            in_specs=[
                pl.BlockSpec((1, C, BH, W),
                             lambda b, h, g=g: (b + g * BG, 0, h, 0)),
                pl.BlockSpec((1, C, BH, W),
                             lambda b, h, g=g: (b + g * BG, 0, h, 0)),
            ],
            out_specs=pl.BlockSpec((1, BH, W), lambda b, h: (b, h, 0)),
            out_shape=jax.ShapeDtypeStruct((BG, H, W), jnp.float32),
        )(pred, target)
        bits = lax.bitcast_convert_type(loss, jnp.int32).reshape(_NW, gchunk)
        h1 = pl.kernel(
            _sc_pass_a,
            out_type=jax.ShapeDtypeStruct((_NW, _NB_A), jnp.int32),
            mesh=mesh,
            compiler_params=scp,
            scratch_types=[
                pltpu.VMEM((gchunk,), jnp.int32),
                pltpu.VMEM(((_NB_A + 16) * _LANES,), jnp.int32),
                pltpu.VMEM((_NB_A,), jnp.int32),
            ],
        )(bits)
        loss_g.append(loss)
        bits_g.append(bits)
        h1_g.append(h1)

    h1 = jnp.concatenate(h1_g, axis=0)
    bits_all = jnp.concatenate(bits_g, axis=1)

    h2, rec1 = pl.kernel(
        functools.partial(_sc_pass_b, k=k),
        out_type=(
            jax.ShapeDtypeStruct((_NW, _NB_B), jnp.int32),
            jax.ShapeDtypeStruct((_LANES,), jnp.int32),
        ),
        mesh=mesh,
        compiler_params=scp,
        scratch_types=[
            pltpu.VMEM((_NG * _NW, _NB_A), jnp.int32),
            pltpu.VMEM((n // _NW // 4,), jnp.int32),
            pltpu.VMEM(((_NB_B + 16) * _LANES,), jnp.int32),
            pltpu.VMEM((_NB_B,), jnp.int32),
            pltpu.VMEM((_LANES,), jnp.int32),
        ],
    )(bits_all, h1)

    hw = (n // _NG) // 1024
    out = pl.pallas_call(
        functools.partial(_finish_body, k=k),
        out_shape=jax.ShapeDtypeStruct((1, 1), jnp.float32),
    )(loss_g[0].reshape(hw, 1024), loss_g[1].reshape(hw, 1024),
      h2, rec1.reshape(1, _LANES))
    return out[0, 0]
